# baseline (speedup 1.0000x reference)
import functools

import jax
import jax.numpy as jnp
from jax import lax
from jax.experimental import pallas as pl
from jax.experimental.pallas import tpu as pltpu
from jax.experimental.pallas import tpu_sc as plsc

DIM = 64
LANES = 16
CHUNK = 128
NBUF = 5
N_WORKERS = 32
HALF0 = 499968

_GATHER_DNUMS = lax.GatherDimensionNumbers(
    offset_dims=(), collapsed_slice_dims=(0,), start_index_map=(0,))


def _bcast_lane(x16, r):
    idx = jnp.full((LANES, 1), r, jnp.int32)
    return lax.gather(x16, idx, _GATHER_DNUMS, (1,),
                      mode=lax.GatherScatterMode.PROMISE_IN_BOUNDS)


def _sc_gather_half(n, vhalf, phase):
    bpw = n // N_WORKERS
    nchunks = bpw // CHUNK
    nrounds = nchunks // NBUF
    mesh = plsc.VectorSubcoreMesh(core_axis_name="c", subcore_axis_name="s")

    scratch = [
        pltpu.VMEM((nchunks, CHUNK), jnp.int32),
        pltpu.VMEM((nchunks, CHUNK), jnp.int32),
        pltpu.VMEM((NBUF, CHUNK, DIM), jnp.float32),
        pltpu.SemaphoreType.DMA((NBUF,)),
        pltpu.SemaphoreType.DMA((NBUF,)),
    ]
    if phase == 1:
        scratch += [
            pltpu.VMEM((NBUF, CHUNK, DIM), jnp.float32),
            pltpu.SemaphoreType.DMA((NBUF,)),
        ]

    @functools.partial(
        pl.kernel,
        mesh=mesh,
        compiler_params=pltpu.CompilerParams(use_tc_tiling_on_sc=False),
        out_type=jax.ShapeDtypeStruct((n, DIM), jnp.float32),
        scratch_types=scratch,
    )
    def body(text_hbm, mask_hbm, table_hbm, *rest):
        if phase == 1:
            prev_hbm, out_hbm, idx_v, fac_v, rows_v, gsem, wsem, prev_v, psem = rest
        else:
            out_hbm, idx_v, fac_v, rows_v, gsem, wsem = rest
            prev_hbm = prev_v = psem = None
        nc = jax.lax.axis_size("c")
        wid = lax.axis_index("s") * nc + lax.axis_index("c")
        base = wid * nchunks
        pltpu.sync_copy(text_hbm.at[pl.ds(base, nchunks)], idx_v)
        pltpu.sync_copy(mask_hbm.at[pl.ds(base, nchunks)], fac_v)
        rbase = wid * bpw

        h0 = jnp.int32(HALF0)

        def compute_chunk(j, _):
            for k in range(CHUNK // LANES):
                sl = pl.ds(k * LANES, LANES)
                t = idx_v[j, sl]
                m = fac_v[j, sl]
                if phase == 0:
                    fac_v[j, sl] = jnp.where(t < h0, m, 0)
                    idx_v[j, sl] = lax.rem(t, h0)
                else:
                    sel = t >= h0
                    fac_v[j, sl] = jnp.where(sel, m, 0)
                    idx_v[j, sl] = jnp.where(sel, t - h0, t)
            return 0

        lax.fori_loop(0, nchunks, compute_chunk, 0)

        def gstart(b, j):
            pltpu.make_async_copy(
                table_hbm.at[idx_v.at[j]], rows_v.at[b], gsem.at[b]).start()

        def gwait(b, j):
            pltpu.make_async_copy(
                table_hbm.at[idx_v.at[j]], rows_v.at[b], gsem.at[b]).wait()

        def pstart(b, j):
            pltpu.make_async_copy(
                prev_hbm.at[pl.ds(rbase + j * CHUNK, CHUNK)], prev_v.at[b],
                psem.at[b]).start()

        def pwait(b, j):
            pltpu.make_async_copy(
                prev_hbm.at[pl.ds(rbase + j * CHUNK, CHUNK)], prev_v.at[b],
                psem.at[b]).wait()

        def wstart(b, j):
            pltpu.make_async_copy(
                rows_v.at[b], out_hbm.at[pl.ds(rbase + j * CHUNK, CHUNK)],
                wsem.at[b]).start()

        def wwait(b, j):
            pltpu.make_async_copy(
                rows_v.at[b], out_hbm.at[pl.ds(rbase + j * CHUNK, CHUNK)],
                wsem.at[b]).wait()

        def mask_rows(b, j):
            def group(g, _):
                f16 = fac_v[j, pl.ds(g * LANES, LANES)].astype(jnp.float32)
                for r in range(LANES):
                    fg = _bcast_lane(f16, r)
                    row = g * LANES + r
                    for k in range(DIM // LANES):
                        sl = pl.ds(k * LANES, LANES)
                        acc = rows_v[b, row, sl] * fg
                        if phase == 1:
                            acc = acc + prev_v[b, row, sl]
                        rows_v[b, row, sl] = acc
                return 0

            lax.fori_loop(0, CHUNK // LANES, group, 0)

        for b in range(NBUF):
            gstart(b, b)
            if phase == 1:
                pstart(b, b)

        def pipeline_round(r, _):
            j0 = r * NBUF
            for b in range(NBUF):
                gwait(b, j0 + b)
                if phase == 1:
                    pwait(b, j0 + b)
                mask_rows(b, j0 + b)
                wstart(b, j0 + b)
            jn0 = j0 + NBUF
            for b in range(NBUF):

                @pl.when(jn0 + b < nchunks)
                def _():
                    wwait(b, j0 + b)
                    gstart(b, jn0 + b)
                    if phase == 1:
                        pstart(b, jn0 + b)

            return 0

        lax.fori_loop(0, nrounds, pipeline_round, 0)
        for b in range(NBUF):
            wwait(b, nchunks - NBUF + b)

    return body


def kernel(text, word_mask, embedding_matrix):
    B, L = text.shape
    n = B * L
    v = embedding_matrix.shape[0]
    text2 = text.reshape(n // CHUNK, CHUNK).astype(jnp.int32)
    mask2 = word_mask.reshape(n // CHUNK, CHUNK).astype(jnp.int32)
    t0 = lax.slice_in_dim(embedding_matrix, 0, HALF0)
    t1 = lax.slice_in_dim(embedding_matrix, HALF0, v)
    out1 = _sc_gather_half(n, HALF0, 0)(text2, mask2, t0)
    out = _sc_gather_half(n, v - HALF0, 1)(text2, mask2, t1, out1)
    return out.reshape(B, L, DIM)


# speedup vs baseline: 1.1484x; 1.1484x over previous
import functools

import jax
import jax.numpy as jnp
from jax import lax
from jax.experimental import pallas as pl
from jax.experimental.pallas import tpu as pltpu
from jax.experimental.pallas import tpu_sc as plsc

DIM = 64
LANES = 16

_GATHER_DNUMS = lax.GatherDimensionNumbers(
    offset_dims=(), collapsed_slice_dims=(0,), start_index_map=(0,))


def _bcast_lane(x16, r):
    idx = jnp.full((LANES, 1), r, jnp.int32)
    return lax.gather(x16, idx, _GATHER_DNUMS, (1,),
                      mode=lax.GatherScatterMode.PROMISE_IN_BOUNDS)
CHUNK = 128
NBUF = 5
N_WORKERS = 32


def _sc_gather(n):
    bpw = n // N_WORKERS
    nchunks = bpw // CHUNK
    nrounds = nchunks // NBUF
    mesh = plsc.VectorSubcoreMesh(core_axis_name="c", subcore_axis_name="s")

    @functools.partial(
        pl.kernel,
        mesh=mesh,
        compiler_params=pltpu.CompilerParams(use_tc_tiling_on_sc=False),
        out_type=jax.ShapeDtypeStruct((n, DIM), jnp.float32),
        scratch_types=[
            pltpu.VMEM((nchunks, CHUNK), jnp.int32),
            pltpu.VMEM((nchunks, CHUNK), jnp.int32),
            pltpu.VMEM((NBUF, CHUNK, DIM), jnp.float32),
            pltpu.SemaphoreType.DMA((NBUF,)),
            pltpu.SemaphoreType.DMA((NBUF,)),
        ],
    )
    def body(text_hbm, mask_hbm, table_hbm, out_hbm, idx_v, mask_v, rows_v, gsem, wsem):
        nc = jax.lax.axis_size("c")
        wid = lax.axis_index("s") * nc + lax.axis_index("c")
        base = wid * nchunks
        pltpu.sync_copy(text_hbm.at[pl.ds(base, nchunks)], idx_v)
        pltpu.sync_copy(mask_hbm.at[pl.ds(base, nchunks)], mask_v)
        rbase = wid * bpw

        def gstart(b, j):
            pltpu.make_async_copy(
                table_hbm.at[idx_v.at[j]], rows_v.at[b], gsem.at[b]).start()

        def gwait(b, j):
            pltpu.make_async_copy(
                table_hbm.at[idx_v.at[j]], rows_v.at[b], gsem.at[b]).wait()

        def wstart(b, j):
            pltpu.make_async_copy(
                rows_v.at[b], out_hbm.at[pl.ds(rbase + j * CHUNK, CHUNK)],
                wsem.at[b]).start()

        def wwait(b, j):
            pltpu.make_async_copy(
                rows_v.at[b], out_hbm.at[pl.ds(rbase + j * CHUNK, CHUNK)],
                wsem.at[b]).wait()

        def mask_rows(b, j):
            def group(g, _):
                m16 = mask_v[j, pl.ds(g * LANES, LANES)].astype(jnp.float32)
                for r in range(LANES):
                    mg = _bcast_lane(m16, r)
                    row = g * LANES + r
                    for k in range(DIM // LANES):
                        sl = pl.ds(k * LANES, LANES)
                        rows_v[b, row, sl] = rows_v[b, row, sl] * mg
                return 0

            lax.fori_loop(0, CHUNK // LANES, group, 0)

        for b in range(NBUF):
            gstart(b, b)

        def pipeline_round(r, _):
            j0 = r * NBUF
            for b in range(NBUF):
                gwait(b, j0 + b)
                mask_rows(b, j0 + b)
                wstart(b, j0 + b)
            jn0 = j0 + NBUF
            for b in range(NBUF):

                @pl.when(jn0 + b < nchunks)
                def _():
                    wwait(b, j0 + b)
                    gstart(b, jn0 + b)

            return 0

        lax.fori_loop(0, nrounds, pipeline_round, 0)
        for b in range(NBUF):
            wwait(b, nchunks - NBUF + b)

    return body


def kernel(text, word_mask, embedding_matrix):
    B, L = text.shape
    n = B * L
    text2 = text.reshape(n // CHUNK, CHUNK).astype(jnp.int32)
    mask2 = word_mask.reshape(n // CHUNK, CHUNK).astype(jnp.int32)
    out = _sc_gather(n)(text2, mask2, embedding_matrix)
    return out.reshape(B, L, DIM)
